# baseline (speedup 1.0000x reference)
import functools

import jax
import jax.numpy as jnp
from jax import lax
from jax.experimental import pallas as pl
from jax.experimental.pallas import tpu as pltpu
from jax.experimental.pallas import tpu_sc as plsc

N_NODES = 10000
N_EDGES = 320000
C_IN = 128

NUM_CORES = 2
NUM_SUBCORES = 16
NW = NUM_CORES * NUM_SUBCORES
LANES = 16

T_SMALL = 78
CH_SMALL = T_SMALL * 128
CH_BIG = CH_SMALL + 128

_mesh = plsc.VectorSubcoreMesh(core_axis_name="c", subcore_axis_name="s")
_sc_params = pltpu.CompilerParams(needs_layout_passes=False)


def _worker_id():
    return lax.axis_index("s") * NUM_CORES + lax.axis_index("c")


def _edge_base(wid):
    return (wid * T_SMALL + jnp.minimum(wid, 4)) * 128


def _copy_edge_block(ei_hbm, e_v, sem, wid):
    base = _edge_base(wid)
    is_big = wid < 4

    @pl.when(is_big)
    def _():
        pltpu.async_copy(ei_hbm.at[:, pl.ds(base, CH_BIG)], e_v, sem).wait()

    @pl.when(jnp.logical_not(is_big))
    def _():
        pltpu.async_copy(
            ei_hbm.at[:, pl.ds(base, CH_SMALL)],
            e_v.at[:, pl.ds(0, CH_SMALL)],
            sem,
        ).wait()


def _zero_vmem(ref, n):
    zeros = jnp.zeros((LANES,), jnp.float32)

    @plsc.parallel_loop(0, n // LANES, unroll=16)
    def _(i):
        ref[pl.ds(i * LANES, LANES)] = zeros


@functools.partial(
    pl.kernel,
    out_type=jax.ShapeDtypeStruct((NW, N_NODES), jnp.float32),
    mesh=_mesh,
    compiler_params=_sc_params,
    scratch_types=[
        pltpu.VMEM((2, CH_BIG), jnp.int32),
        pltpu.VMEM((N_NODES,), jnp.float32),
        pltpu.SemaphoreType.DMA,
    ],
)
def _deg_kernel(ei_hbm, out_hbm, e_v, acc_v, sem):
    wid = _worker_id()
    _zero_vmem(acc_v, N_NODES)
    _copy_edge_block(ei_hbm, e_v, sem, wid)
    ones = jnp.ones((LANES,), jnp.float32)

    @plsc.parallel_loop(0, CH_SMALL // LANES, unroll=16)
    def _(i):
        idx = e_v[1, pl.ds(i * LANES, LANES)]
        plsc.addupdate_scatter(acc_v, [idx], ones)

    @pl.when(wid < 4)
    def _():
        @plsc.parallel_loop(CH_SMALL // LANES, CH_BIG // LANES, unroll=16)
        def _(i):
            idx = e_v[1, pl.ds(i * LANES, LANES)]
            plsc.addupdate_scatter(acc_v, [idx], ones)

    pltpu.sync_copy(acc_v, out_hbm.at[wid])


@functools.partial(
    pl.kernel,
    out_type=jax.ShapeDtypeStruct((NW, N_NODES), jnp.float32),
    mesh=_mesh,
    compiler_params=_sc_params,
    scratch_types=[
        pltpu.VMEM((2, CH_BIG), jnp.int32),
        pltpu.VMEM((N_NODES,), jnp.float32),
        pltpu.VMEM((N_NODES,), jnp.float32),
        pltpu.SemaphoreType.DMA,
        pltpu.SemaphoreType.DMA,
    ],
)
def _msg_kernel(ei_hbm, g_hbm, out_hbm, e_v, g_v, acc_v, sem_e, sem_g):
    wid = _worker_id()
    cpg = pltpu.async_copy(
        g_hbm.at[pl.ds(lax.rem(wid, 4) * N_NODES, N_NODES)], g_v, sem_g
    )
    _zero_vmem(acc_v, N_NODES)
    _copy_edge_block(ei_hbm, e_v, sem_e, wid)
    cpg.wait()

    @plsc.parallel_loop(0, CH_SMALL // LANES, unroll=16)
    def _(i):
        isrc = e_v[0, pl.ds(i * LANES, LANES)]
        idst = e_v[1, pl.ds(i * LANES, LANES)]
        vals = plsc.load_gather(g_v, [isrc])
        plsc.addupdate_scatter(acc_v, [idst], vals)

    @pl.when(wid < 4)
    def _():
        @plsc.parallel_loop(CH_SMALL // LANES, CH_BIG // LANES, unroll=16)
        def _(i):
            isrc = e_v[0, pl.ds(i * LANES, LANES)]
            idst = e_v[1, pl.ds(i * LANES, LANES)]
            vals = plsc.load_gather(g_v, [isrc])
            plsc.addupdate_scatter(acc_v, [idst], vals)

    pltpu.sync_copy(acc_v, out_hbm.at[wid])


def _h_body(x_ref, w_ref, h_ref):
    z = lax.dot_general(
        w_ref[...], x_ref[...], (((1,), (1,)), ((), ())),
        preferred_element_type=jnp.float32,
    )
    h_ref[...] = z[0]


def _mid_body(degp_ref, h_ref, g_ref, dis_ref):
    deg = jnp.sum(degp_ref[...], axis=0) + 1.0
    dis = lax.rsqrt(deg)
    dis_ref[...] = dis
    g = h_ref[...] * dis
    for r in range(4):
        g_ref[pl.ds(r * N_NODES, N_NODES)] = g


def _fin_body(msgp_ref, g_ref, dis_ref, b_ref, out_ref):
    p = jnp.sum(msgp_ref[...], axis=0)
    z = dis_ref[...] * (p + g_ref[pl.ds(0, N_NODES)]) + b_ref[...]
    out_ref[...] = jax.nn.sigmoid(z)[:, None]


def kernel(x, edge_index, W, b):
    ei = edge_index.astype(jnp.int32)
    w2d = W.reshape((1, C_IN)).astype(jnp.float32)

    degp = _deg_kernel(ei)

    h = pl.pallas_call(
        _h_body,
        out_shape=jax.ShapeDtypeStruct((N_NODES,), jnp.float32),
    )(x, w2d)

    g4, dis = pl.pallas_call(
        _mid_body,
        out_shape=(
            jax.ShapeDtypeStruct((4 * N_NODES,), jnp.float32),
            jax.ShapeDtypeStruct((N_NODES,), jnp.float32),
        ),
    )(degp, h)

    msgp = _msg_kernel(ei, g4)

    out = pl.pallas_call(
        _fin_body,
        out_shape=jax.ShapeDtypeStruct((N_NODES, 1), jnp.float32),
    )(msgp, g4, dis, b)

    return out


# speedup vs baseline: 1.0030x; 1.0030x over previous
import functools

import jax
import jax.numpy as jnp
from jax import lax
from jax.experimental import pallas as pl
from jax.experimental.pallas import tpu as pltpu
from jax.experimental.pallas import tpu_sc as plsc

N_NODES = 10000
N_EDGES = 320000
C_IN = 128

NUM_CORES = 2
NUM_SUBCORES = 16
NW = NUM_CORES * NUM_SUBCORES
LANES = 16

T_SMALL = 78
CH_SMALL = T_SMALL * 128
CH_BIG = CH_SMALL + 128

_mesh = plsc.VectorSubcoreMesh(core_axis_name="c", subcore_axis_name="s")
_sc_params = pltpu.CompilerParams(needs_layout_passes=False)


def _worker_id():
    return lax.axis_index("s") * NUM_CORES + lax.axis_index("c")


def _edge_base(wid):
    return (wid * T_SMALL + jnp.minimum(wid, 4)) * 128


def _copy_edge_block(ei_hbm, e_v, sem, wid):
    base = _edge_base(wid)
    is_big = wid < 4

    @pl.when(is_big)
    def _():
        pltpu.async_copy(ei_hbm.at[:, pl.ds(base, CH_BIG)], e_v, sem).wait()

    @pl.when(jnp.logical_not(is_big))
    def _():
        pltpu.async_copy(
            ei_hbm.at[:, pl.ds(base, CH_SMALL)],
            e_v.at[:, pl.ds(0, CH_SMALL)],
            sem,
        ).wait()


def _zero_vmem(ref, n):
    zeros = jnp.zeros((LANES,), jnp.float32)

    @plsc.parallel_loop(0, n // LANES, unroll=8)
    def _(i):
        ref[pl.ds(i * LANES, LANES)] = zeros


@functools.partial(
    pl.kernel,
    out_type=jax.ShapeDtypeStruct((NW, N_NODES), jnp.float32),
    mesh=_mesh,
    compiler_params=_sc_params,
    scratch_types=[
        pltpu.VMEM((2, CH_BIG), jnp.int32),
        pltpu.VMEM((N_NODES,), jnp.float32),
        pltpu.SemaphoreType.DMA,
    ],
)
def _deg_kernel(ei_hbm, out_hbm, e_v, acc_v, sem):
    wid = _worker_id()
    _zero_vmem(acc_v, N_NODES)
    _copy_edge_block(ei_hbm, e_v, sem, wid)
    ones = jnp.ones((LANES,), jnp.float32)

    @plsc.parallel_loop(0, CH_SMALL // LANES, unroll=8)
    def _(i):
        idx = e_v[1, pl.ds(i * LANES, LANES)]
        plsc.addupdate_scatter(acc_v, [idx], ones)

    @pl.when(wid < 4)
    def _():
        @plsc.parallel_loop(CH_SMALL // LANES, CH_BIG // LANES, unroll=8)
        def _(i):
            idx = e_v[1, pl.ds(i * LANES, LANES)]
            plsc.addupdate_scatter(acc_v, [idx], ones)

    pltpu.sync_copy(acc_v, out_hbm.at[wid])


@functools.partial(
    pl.kernel,
    out_type=jax.ShapeDtypeStruct((NW, N_NODES), jnp.float32),
    mesh=_mesh,
    compiler_params=_sc_params,
    scratch_types=[
        pltpu.VMEM((2, CH_BIG), jnp.int32),
        pltpu.VMEM((N_NODES,), jnp.float32),
        pltpu.VMEM((N_NODES,), jnp.float32),
        pltpu.SemaphoreType.DMA,
        pltpu.SemaphoreType.DMA,
    ],
)
def _msg_kernel(ei_hbm, g_hbm, out_hbm, e_v, g_v, acc_v, sem_e, sem_g):
    wid = _worker_id()
    cpg = pltpu.async_copy(
        g_hbm.at[pl.ds(lax.rem(wid, 4) * N_NODES, N_NODES)], g_v, sem_g
    )
    _zero_vmem(acc_v, N_NODES)
    _copy_edge_block(ei_hbm, e_v, sem_e, wid)
    cpg.wait()

    @plsc.parallel_loop(0, CH_SMALL // LANES, unroll=8)
    def _(i):
        isrc = e_v[0, pl.ds(i * LANES, LANES)]
        idst = e_v[1, pl.ds(i * LANES, LANES)]
        vals = plsc.load_gather(g_v, [isrc])
        plsc.addupdate_scatter(acc_v, [idst], vals)

    @pl.when(wid < 4)
    def _():
        @plsc.parallel_loop(CH_SMALL // LANES, CH_BIG // LANES, unroll=8)
        def _(i):
            isrc = e_v[0, pl.ds(i * LANES, LANES)]
            idst = e_v[1, pl.ds(i * LANES, LANES)]
            vals = plsc.load_gather(g_v, [isrc])
            plsc.addupdate_scatter(acc_v, [idst], vals)

    pltpu.sync_copy(acc_v, out_hbm.at[wid])


def _h_body(x_ref, w_ref, h_ref):
    z = lax.dot_general(
        w_ref[...], x_ref[...], (((1,), (1,)), ((), ())),
        preferred_element_type=jnp.float32,
    )
    h_ref[...] = z[0]


def _mid_body(degp_ref, h_ref, g_ref, dis_ref):
    deg = jnp.sum(degp_ref[...], axis=0) + 1.0
    dis = lax.rsqrt(deg)
    dis_ref[...] = dis
    g = h_ref[...] * dis
    for r in range(4):
        g_ref[pl.ds(r * N_NODES, N_NODES)] = g


def _fin_body(msgp_ref, g_ref, dis_ref, b_ref, out_ref):
    p = jnp.sum(msgp_ref[...], axis=0)
    z = dis_ref[...] * (p + g_ref[pl.ds(0, N_NODES)]) + b_ref[...]
    out_ref[...] = jax.nn.sigmoid(z)[:, None]


def kernel(x, edge_index, W, b):
    ei = edge_index.astype(jnp.int32)
    w2d = W.reshape((1, C_IN)).astype(jnp.float32)

    degp = _deg_kernel(ei)

    h = pl.pallas_call(
        _h_body,
        out_shape=jax.ShapeDtypeStruct((N_NODES,), jnp.float32),
    )(x, w2d)

    g4, dis = pl.pallas_call(
        _mid_body,
        out_shape=(
            jax.ShapeDtypeStruct((4 * N_NODES,), jnp.float32),
            jax.ShapeDtypeStruct((N_NODES,), jnp.float32),
        ),
    )(degp, h)

    msgp = _msg_kernel(ei, g4)

    out = pl.pallas_call(
        _fin_body,
        out_shape=jax.ShapeDtypeStruct((N_NODES, 1), jnp.float32),
    )(msgp, g4, dis, b)

    return out


# speedup vs baseline: 1.1549x; 1.1515x over previous
import functools

import jax
import jax.numpy as jnp
from jax import lax
from jax.experimental import pallas as pl
from jax.experimental.pallas import tpu as pltpu
from jax.experimental.pallas import tpu_sc as plsc

N_NODES = 10000
N_EDGES = 320000
C_IN = 128

NUM_CORES = 2
NUM_SUBCORES = 16
NW = NUM_CORES * NUM_SUBCORES
LANES = 16

T_SMALL = 78
CH_SMALL = T_SMALL * 128
CH_BIG = CH_SMALL + 128

_mesh = plsc.VectorSubcoreMesh(core_axis_name="c", subcore_axis_name="s")
_sc_params = pltpu.CompilerParams(needs_layout_passes=False)


def _worker_id():
    return lax.axis_index("s") * NUM_CORES + lax.axis_index("c")


def _edge_base(wid):
    return (wid * T_SMALL + jnp.minimum(wid, 4)) * 128


def _copy_edge_block(ei_hbm, e_v, sem, wid):
    base = _edge_base(wid)
    is_big = wid < 4

    @pl.when(is_big)
    def _():
        pltpu.async_copy(ei_hbm.at[:, pl.ds(base, CH_BIG)], e_v, sem).wait()

    @pl.when(jnp.logical_not(is_big))
    def _():
        pltpu.async_copy(
            ei_hbm.at[:, pl.ds(base, CH_SMALL)],
            e_v.at[:, pl.ds(0, CH_SMALL)],
            sem,
        ).wait()


def _zero_vmem(ref, n):
    zeros = jnp.zeros((LANES,), jnp.float32)

    @plsc.parallel_loop(0, n // LANES, unroll=8)
    def _(i):
        ref[pl.ds(i * LANES, LANES)] = zeros


@functools.partial(
    pl.kernel,
    out_type=jax.ShapeDtypeStruct((NW, N_NODES), jnp.float32),
    mesh=_mesh,
    compiler_params=_sc_params,
    scratch_types=[
        pltpu.VMEM((2, CH_BIG), jnp.int32),
        pltpu.VMEM((N_NODES,), jnp.float32),
        pltpu.SemaphoreType.DMA,
    ],
)
def _deg_kernel(ei_hbm, out_hbm, e_v, acc_v, sem):
    wid = _worker_id()
    _zero_vmem(acc_v, N_NODES)
    _copy_edge_block(ei_hbm, e_v, sem, wid)
    ones = jnp.ones((LANES,), jnp.float32)

    @plsc.parallel_loop(0, CH_SMALL // LANES, unroll=8)
    def _(i):
        idx = e_v[1, pl.ds(i * LANES, LANES)]
        plsc.addupdate_scatter(acc_v, [idx], ones)

    @pl.when(wid < 4)
    def _():
        @plsc.parallel_loop(CH_SMALL // LANES, CH_BIG // LANES, unroll=8)
        def _(i):
            idx = e_v[1, pl.ds(i * LANES, LANES)]
            plsc.addupdate_scatter(acc_v, [idx], ones)

    pltpu.sync_copy(acc_v, out_hbm.at[wid])


@functools.partial(
    pl.kernel,
    out_type=jax.ShapeDtypeStruct((NW, N_NODES), jnp.float32),
    mesh=_mesh,
    compiler_params=_sc_params,
    scratch_types=[
        pltpu.VMEM((2, CH_BIG), jnp.int32),
        pltpu.VMEM((N_NODES,), jnp.float32),
        pltpu.VMEM((N_NODES,), jnp.float32),
        pltpu.SemaphoreType.DMA,
        pltpu.SemaphoreType.DMA,
    ],
)
def _msg_kernel(ei_hbm, g_hbm, out_hbm, e_v, g_v, acc_v, sem_e, sem_g):
    wid = _worker_id()
    cpg = pltpu.async_copy(
        g_hbm.at[pl.ds(lax.rem(wid, 4) * N_NODES, N_NODES)], g_v, sem_g
    )
    _zero_vmem(acc_v, N_NODES)
    _copy_edge_block(ei_hbm, e_v, sem_e, wid)
    cpg.wait()

    @plsc.parallel_loop(0, CH_SMALL // LANES, unroll=8)
    def _(i):
        isrc = e_v[0, pl.ds(i * LANES, LANES)]
        idst = e_v[1, pl.ds(i * LANES, LANES)]
        vals = plsc.load_gather(g_v, [isrc])
        plsc.addupdate_scatter(acc_v, [idst], vals)

    @pl.when(wid < 4)
    def _():
        @plsc.parallel_loop(CH_SMALL // LANES, CH_BIG // LANES, unroll=8)
        def _(i):
            isrc = e_v[0, pl.ds(i * LANES, LANES)]
            idst = e_v[1, pl.ds(i * LANES, LANES)]
            vals = plsc.load_gather(g_v, [isrc])
            plsc.addupdate_scatter(acc_v, [idst], vals)

    pltpu.sync_copy(acc_v, out_hbm.at[wid])


def _h_body(x_ref, w_ref, h_ref):
    z = lax.dot_general(
        w_ref[...], x_ref[...], (((1,), (1,)), ((), ())),
        preferred_element_type=jnp.float32,
    )
    h_ref[...] = z[0]


def _mid_body(degp_ref, h_ref, g_ref, dis_ref):
    deg = jnp.sum(degp_ref[...], axis=0) + 1.0
    dis = lax.rsqrt(deg)
    dis_ref[...] = dis
    g = h_ref[...] * dis
    for r in range(4):
        g_ref[pl.ds(r * N_NODES, N_NODES)] = g


def _fin_body(msgp_ref, g_ref, dis_ref, b_ref, out_ref):
    p = jnp.sum(msgp_ref[...], axis=0)
    z = dis_ref[...] * (p + g_ref[pl.ds(0, N_NODES)]) + b_ref[...]
    out_ref[...] = jax.nn.sigmoid(z)


def kernel(x, edge_index, W, b):
    ei = edge_index.astype(jnp.int32)
    w2d = W.reshape((1, C_IN)).astype(jnp.float32)

    degp = _deg_kernel(ei)

    h = pl.pallas_call(
        _h_body,
        out_shape=jax.ShapeDtypeStruct((N_NODES,), jnp.float32),
    )(x, w2d)

    g4, dis = pl.pallas_call(
        _mid_body,
        out_shape=(
            jax.ShapeDtypeStruct((4 * N_NODES,), jnp.float32),
            jax.ShapeDtypeStruct((N_NODES,), jnp.float32),
        ),
    )(degp, h)

    msgp = _msg_kernel(ei, g4)

    out = pl.pallas_call(
        _fin_body,
        out_shape=jax.ShapeDtypeStruct((N_NODES,), jnp.float32),
    )(msgp, g4, dis, b)

    return out[:, None]


# speedup vs baseline: 1.1550x; 1.0001x over previous
import functools

import jax
import jax.numpy as jnp
from jax import lax
from jax.experimental import pallas as pl
from jax.experimental.pallas import tpu as pltpu
from jax.experimental.pallas import tpu_sc as plsc

N_NODES = 10000
N_EDGES = 320000
C_IN = 128

NUM_CORES = 2
NUM_SUBCORES = 16
NW = NUM_CORES * NUM_SUBCORES
LANES = 16

T_SMALL = 78
CH_SMALL = T_SMALL * 128
CH_BIG = CH_SMALL + 128

_mesh = plsc.VectorSubcoreMesh(core_axis_name="c", subcore_axis_name="s")
_sc_params = pltpu.CompilerParams(
    needs_layout_passes=False, skip_device_barrier=True
)
_tc_params = pltpu.CompilerParams(skip_device_barrier=True)


def _worker_id():
    return lax.axis_index("s") * NUM_CORES + lax.axis_index("c")


def _edge_base(wid):
    return (wid * T_SMALL + jnp.minimum(wid, 4)) * 128


def _copy_edge_block(ei_hbm, e_v, sem, wid):
    base = _edge_base(wid)
    is_big = wid < 4

    @pl.when(is_big)
    def _():
        pltpu.async_copy(ei_hbm.at[:, pl.ds(base, CH_BIG)], e_v, sem).wait()

    @pl.when(jnp.logical_not(is_big))
    def _():
        pltpu.async_copy(
            ei_hbm.at[:, pl.ds(base, CH_SMALL)],
            e_v.at[:, pl.ds(0, CH_SMALL)],
            sem,
        ).wait()


def _zero_vmem(ref, n):
    zeros = jnp.zeros((LANES,), jnp.float32)

    @plsc.parallel_loop(0, n // LANES, unroll=8)
    def _(i):
        ref[pl.ds(i * LANES, LANES)] = zeros


@functools.partial(
    pl.kernel,
    out_type=jax.ShapeDtypeStruct((NW, N_NODES), jnp.float32),
    mesh=_mesh,
    compiler_params=_sc_params,
    scratch_types=[
        pltpu.VMEM((2, CH_BIG), jnp.int32),
        pltpu.VMEM((N_NODES,), jnp.float32),
        pltpu.SemaphoreType.DMA,
    ],
)
def _deg_kernel(ei_hbm, out_hbm, e_v, acc_v, sem):
    wid = _worker_id()
    _zero_vmem(acc_v, N_NODES)
    _copy_edge_block(ei_hbm, e_v, sem, wid)
    ones = jnp.ones((LANES,), jnp.float32)

    @plsc.parallel_loop(0, CH_SMALL // LANES, unroll=8)
    def _(i):
        idx = e_v[1, pl.ds(i * LANES, LANES)]
        plsc.addupdate_scatter(acc_v, [idx], ones)

    @pl.when(wid < 4)
    def _():
        @plsc.parallel_loop(CH_SMALL // LANES, CH_BIG // LANES, unroll=8)
        def _(i):
            idx = e_v[1, pl.ds(i * LANES, LANES)]
            plsc.addupdate_scatter(acc_v, [idx], ones)

    pltpu.sync_copy(acc_v, out_hbm.at[wid])


@functools.partial(
    pl.kernel,
    out_type=jax.ShapeDtypeStruct((NW, N_NODES), jnp.float32),
    mesh=_mesh,
    compiler_params=_sc_params,
    scratch_types=[
        pltpu.VMEM((2, CH_BIG), jnp.int32),
        pltpu.VMEM((N_NODES,), jnp.float32),
        pltpu.VMEM((N_NODES,), jnp.float32),
        pltpu.SemaphoreType.DMA,
        pltpu.SemaphoreType.DMA,
    ],
)
def _msg_kernel(ei_hbm, g_hbm, out_hbm, e_v, g_v, acc_v, sem_e, sem_g):
    wid = _worker_id()
    cpg = pltpu.async_copy(
        g_hbm.at[pl.ds(lax.rem(wid, 4) * N_NODES, N_NODES)], g_v, sem_g
    )
    _zero_vmem(acc_v, N_NODES)
    _copy_edge_block(ei_hbm, e_v, sem_e, wid)
    cpg.wait()

    @plsc.parallel_loop(0, CH_SMALL // LANES, unroll=8)
    def _(i):
        isrc = e_v[0, pl.ds(i * LANES, LANES)]
        idst = e_v[1, pl.ds(i * LANES, LANES)]
        vals = plsc.load_gather(g_v, [isrc])
        plsc.addupdate_scatter(acc_v, [idst], vals)

    @pl.when(wid < 4)
    def _():
        @plsc.parallel_loop(CH_SMALL // LANES, CH_BIG // LANES, unroll=8)
        def _(i):
            isrc = e_v[0, pl.ds(i * LANES, LANES)]
            idst = e_v[1, pl.ds(i * LANES, LANES)]
            vals = plsc.load_gather(g_v, [isrc])
            plsc.addupdate_scatter(acc_v, [idst], vals)

    pltpu.sync_copy(acc_v, out_hbm.at[wid])


def _h_body(x_ref, w_ref, h_ref):
    z = lax.dot_general(
        w_ref[...], x_ref[...], (((1,), (1,)), ((), ())),
        preferred_element_type=jnp.float32,
    )
    h_ref[...] = z[0]


def _mid_body(degp_ref, h_ref, g_ref, dis_ref):
    deg = jnp.sum(degp_ref[...], axis=0) + 1.0
    dis = lax.rsqrt(deg)
    dis_ref[...] = dis
    g = h_ref[...] * dis
    for r in range(4):
        g_ref[pl.ds(r * N_NODES, N_NODES)] = g


def _fin_body(msgp_ref, g_ref, dis_ref, b_ref, out_ref):
    p = jnp.sum(msgp_ref[...], axis=0)
    z = dis_ref[...] * (p + g_ref[pl.ds(0, N_NODES)]) + b_ref[...]
    out_ref[...] = jax.nn.sigmoid(z)


def kernel(x, edge_index, W, b):
    ei = edge_index.astype(jnp.int32)
    w2d = W.reshape((1, C_IN)).astype(jnp.float32)

    degp = _deg_kernel(ei)

    h = pl.pallas_call(
        _h_body,
        out_shape=jax.ShapeDtypeStruct((N_NODES,), jnp.float32),
        compiler_params=_tc_params,
    )(x, w2d)

    g4, dis = pl.pallas_call(
        _mid_body,
        out_shape=(
            jax.ShapeDtypeStruct((4 * N_NODES,), jnp.float32),
            jax.ShapeDtypeStruct((N_NODES,), jnp.float32),
        ),
        compiler_params=_tc_params,
    )(degp, h)

    msgp = _msg_kernel(ei, g4)

    out = pl.pallas_call(
        _fin_body,
        out_shape=jax.ShapeDtypeStruct((N_NODES,), jnp.float32),
        compiler_params=_tc_params,
    )(msgp, g4, dis, b)

    return out[:, None]


# speedup vs baseline: 1.1656x; 1.0092x over previous
import functools

import jax
import jax.numpy as jnp
from jax import lax
from jax.experimental import pallas as pl
from jax.experimental.pallas import tpu as pltpu
from jax.experimental.pallas import tpu_sc as plsc

N_NODES = 10000
N_EDGES = 320000
C_IN = 128

NUM_CORES = 2
NUM_SUBCORES = 16
NW = NUM_CORES * NUM_SUBCORES
LANES = 16

T_SMALL = 78
CH_SMALL = T_SMALL * 128
CH_BIG = CH_SMALL + 128

_mesh = plsc.VectorSubcoreMesh(core_axis_name="c", subcore_axis_name="s")
_sc_params = pltpu.CompilerParams(needs_layout_passes=False)


def _worker_id():
    return lax.axis_index("s") * NUM_CORES + lax.axis_index("c")


def _edge_base(wid):
    return (wid * T_SMALL + jnp.minimum(wid, 4)) * 128


def _start_edge_copy(ei_hbm, e_v, sem, wid):
    base = _edge_base(wid)
    is_big = wid < 4

    @pl.when(is_big)
    def _():
        pltpu.async_copy(ei_hbm.at[:, pl.ds(base, CH_BIG)], e_v, sem)

    @pl.when(jnp.logical_not(is_big))
    def _():
        pltpu.async_copy(
            ei_hbm.at[:, pl.ds(base, CH_SMALL)],
            e_v.at[:, pl.ds(0, CH_SMALL)],
            sem,
        )


def _wait_edge_copy(ei_hbm, e_v, sem, wid):
    base = _edge_base(wid)
    is_big = wid < 4

    @pl.when(is_big)
    def _():
        pltpu.make_async_copy(
            ei_hbm.at[:, pl.ds(base, CH_BIG)], e_v, sem
        ).wait()

    @pl.when(jnp.logical_not(is_big))
    def _():
        pltpu.make_async_copy(
            ei_hbm.at[:, pl.ds(base, CH_SMALL)],
            e_v.at[:, pl.ds(0, CH_SMALL)],
            sem,
        ).wait()


def _zero_vmem(ref, n):
    zeros = jnp.zeros((LANES,), jnp.float32)

    @plsc.parallel_loop(0, n // LANES, unroll=8)
    def _(i):
        ref[pl.ds(i * LANES, LANES)] = zeros


@functools.partial(
    pl.kernel,
    out_type=jax.ShapeDtypeStruct((NW, N_NODES), jnp.float32),
    mesh=_mesh,
    compiler_params=_sc_params,
    scratch_types=[
        pltpu.VMEM((2, CH_BIG), jnp.int32),
        pltpu.VMEM((N_NODES,), jnp.float32),
        pltpu.SemaphoreType.DMA,
    ],
)
def _deg_kernel(ei_hbm, out_hbm, e_v, acc_v, sem):
    wid = _worker_id()
    _start_edge_copy(ei_hbm, e_v, sem, wid)
    _zero_vmem(acc_v, N_NODES)
    _wait_edge_copy(ei_hbm, e_v, sem, wid)
    ones = jnp.ones((LANES,), jnp.float32)

    @plsc.parallel_loop(0, CH_SMALL // LANES, unroll=8)
    def _(i):
        idx = e_v[1, pl.ds(i * LANES, LANES)]
        plsc.addupdate_scatter(acc_v, [idx], ones)

    @pl.when(wid < 4)
    def _():
        @plsc.parallel_loop(CH_SMALL // LANES, CH_BIG // LANES, unroll=8)
        def _(i):
            idx = e_v[1, pl.ds(i * LANES, LANES)]
            plsc.addupdate_scatter(acc_v, [idx], ones)

    pltpu.sync_copy(acc_v, out_hbm.at[wid])


@functools.partial(
    pl.kernel,
    out_type=jax.ShapeDtypeStruct((NW, N_NODES), jnp.float32),
    mesh=_mesh,
    compiler_params=_sc_params,
    scratch_types=[
        pltpu.VMEM((2, CH_BIG), jnp.int32),
        pltpu.VMEM((N_NODES,), jnp.float32),
        pltpu.VMEM((N_NODES,), jnp.float32),
        pltpu.SemaphoreType.DMA,
        pltpu.SemaphoreType.DMA,
    ],
)
def _msg_kernel(ei_hbm, g_hbm, out_hbm, e_v, g_v, acc_v, sem_e, sem_g):
    wid = _worker_id()
    cpg = pltpu.async_copy(
        g_hbm.at[pl.ds(lax.rem(wid, 4) * N_NODES, N_NODES)], g_v, sem_g
    )
    _start_edge_copy(ei_hbm, e_v, sem_e, wid)
    _zero_vmem(acc_v, N_NODES)
    cpg.wait()
    _wait_edge_copy(ei_hbm, e_v, sem_e, wid)

    @plsc.parallel_loop(0, CH_SMALL // LANES, unroll=8)
    def _(i):
        isrc = e_v[0, pl.ds(i * LANES, LANES)]
        idst = e_v[1, pl.ds(i * LANES, LANES)]
        vals = plsc.load_gather(g_v, [isrc])
        plsc.addupdate_scatter(acc_v, [idst], vals)

    @pl.when(wid < 4)
    def _():
        @plsc.parallel_loop(CH_SMALL // LANES, CH_BIG // LANES, unroll=8)
        def _(i):
            isrc = e_v[0, pl.ds(i * LANES, LANES)]
            idst = e_v[1, pl.ds(i * LANES, LANES)]
            vals = plsc.load_gather(g_v, [isrc])
            plsc.addupdate_scatter(acc_v, [idst], vals)

    pltpu.sync_copy(acc_v, out_hbm.at[wid])


def _h_body(x_ref, w_ref, h_ref):
    z = lax.dot_general(
        w_ref[...], x_ref[...], (((1,), (1,)), ((), ())),
        preferred_element_type=jnp.float32,
    )
    h_ref[...] = z[0]


def _mid_body(degp_ref, h_ref, g_ref, dis_ref):
    deg = jnp.sum(degp_ref[...], axis=0) + 1.0
    dis = lax.rsqrt(deg)
    dis_ref[...] = dis
    g = h_ref[...] * dis
    for r in range(4):
        g_ref[pl.ds(r * N_NODES, N_NODES)] = g


def _fin_body(msgp_ref, g_ref, dis_ref, b_ref, out_ref):
    p = jnp.sum(msgp_ref[...], axis=0)
    z = dis_ref[...] * (p + g_ref[pl.ds(0, N_NODES)]) + b_ref[...]
    out_ref[...] = jax.nn.sigmoid(z)


def kernel(x, edge_index, W, b):
    ei = edge_index.astype(jnp.int32)
    w2d = W.reshape((1, C_IN)).astype(jnp.float32)

    degp = _deg_kernel(ei)

    h = pl.pallas_call(
        _h_body,
        out_shape=jax.ShapeDtypeStruct((N_NODES,), jnp.float32),
    )(x, w2d)

    g4, dis = pl.pallas_call(
        _mid_body,
        out_shape=(
            jax.ShapeDtypeStruct((4 * N_NODES,), jnp.float32),
            jax.ShapeDtypeStruct((N_NODES,), jnp.float32),
        ),
    )(degp, h)

    msgp = _msg_kernel(ei, g4)

    out = pl.pallas_call(
        _fin_body,
        out_shape=jax.ShapeDtypeStruct((N_NODES,), jnp.float32),
    )(msgp, g4, dis, b)

    return out[:, None]


# speedup vs baseline: 1.1674x; 1.0015x over previous
import functools

import jax
import jax.numpy as jnp
from jax import lax
from jax.experimental import pallas as pl
from jax.experimental.pallas import tpu as pltpu
from jax.experimental.pallas import tpu_sc as plsc

N_NODES = 10000
N_EDGES = 320000
C_IN = 128

NUM_CORES = 2
NUM_SUBCORES = 16
NW = NUM_CORES * NUM_SUBCORES
LANES = 16

T_SMALL = 78
CH_SMALL = T_SMALL * 128
CH_BIG = CH_SMALL + 128

_mesh = plsc.VectorSubcoreMesh(core_axis_name="c", subcore_axis_name="s")
_sc_params = pltpu.CompilerParams(needs_layout_passes=False)


def _worker_id():
    return lax.axis_index("s") * NUM_CORES + lax.axis_index("c")


def _edge_base(wid):
    return (wid * T_SMALL + jnp.minimum(wid, 4)) * 128


def _start_edge_copy(ei_hbm, e_v, sem, wid):
    base = _edge_base(wid)
    is_big = wid < 4

    @pl.when(is_big)
    def _():
        pltpu.async_copy(ei_hbm.at[:, pl.ds(base, CH_BIG)], e_v, sem)

    @pl.when(jnp.logical_not(is_big))
    def _():
        pltpu.async_copy(
            ei_hbm.at[:, pl.ds(base, CH_SMALL)],
            e_v.at[:, pl.ds(0, CH_SMALL)],
            sem,
        )


def _wait_edge_copy(ei_hbm, e_v, sem, wid):
    base = _edge_base(wid)
    is_big = wid < 4

    @pl.when(is_big)
    def _():
        pltpu.make_async_copy(
            ei_hbm.at[:, pl.ds(base, CH_BIG)], e_v, sem
        ).wait()

    @pl.when(jnp.logical_not(is_big))
    def _():
        pltpu.make_async_copy(
            ei_hbm.at[:, pl.ds(base, CH_SMALL)],
            e_v.at[:, pl.ds(0, CH_SMALL)],
            sem,
        ).wait()


def _zero_vmem(ref, n):
    zeros = jnp.zeros((LANES,), jnp.float32)

    @plsc.parallel_loop(0, n // LANES, unroll=4)
    def _(i):
        ref[pl.ds(i * LANES, LANES)] = zeros


@functools.partial(
    pl.kernel,
    out_type=jax.ShapeDtypeStruct((NW, N_NODES), jnp.float32),
    mesh=_mesh,
    compiler_params=_sc_params,
    scratch_types=[
        pltpu.VMEM((2, CH_BIG), jnp.int32),
        pltpu.VMEM((N_NODES,), jnp.float32),
        pltpu.SemaphoreType.DMA,
    ],
)
def _deg_kernel(ei_hbm, out_hbm, e_v, acc_v, sem):
    wid = _worker_id()
    _start_edge_copy(ei_hbm, e_v, sem, wid)
    _zero_vmem(acc_v, N_NODES)
    _wait_edge_copy(ei_hbm, e_v, sem, wid)
    ones = jnp.ones((LANES,), jnp.float32)

    @plsc.parallel_loop(0, CH_SMALL // LANES, unroll=4)
    def _(i):
        idx = e_v[1, pl.ds(i * LANES, LANES)]
        plsc.addupdate_scatter(acc_v, [idx], ones)

    @pl.when(wid < 4)
    def _():
        @plsc.parallel_loop(CH_SMALL // LANES, CH_BIG // LANES, unroll=4)
        def _(i):
            idx = e_v[1, pl.ds(i * LANES, LANES)]
            plsc.addupdate_scatter(acc_v, [idx], ones)

    pltpu.sync_copy(acc_v, out_hbm.at[wid])


@functools.partial(
    pl.kernel,
    out_type=jax.ShapeDtypeStruct((NW, N_NODES), jnp.float32),
    mesh=_mesh,
    compiler_params=_sc_params,
    scratch_types=[
        pltpu.VMEM((2, CH_BIG), jnp.int32),
        pltpu.VMEM((N_NODES,), jnp.float32),
        pltpu.VMEM((N_NODES,), jnp.float32),
        pltpu.SemaphoreType.DMA,
        pltpu.SemaphoreType.DMA,
    ],
)
def _msg_kernel(ei_hbm, g_hbm, out_hbm, e_v, g_v, acc_v, sem_e, sem_g):
    wid = _worker_id()
    cpg = pltpu.async_copy(
        g_hbm.at[pl.ds(lax.rem(wid, 4) * N_NODES, N_NODES)], g_v, sem_g
    )
    _start_edge_copy(ei_hbm, e_v, sem_e, wid)
    _zero_vmem(acc_v, N_NODES)
    cpg.wait()
    _wait_edge_copy(ei_hbm, e_v, sem_e, wid)

    @plsc.parallel_loop(0, CH_SMALL // LANES, unroll=4)
    def _(i):
        isrc = e_v[0, pl.ds(i * LANES, LANES)]
        idst = e_v[1, pl.ds(i * LANES, LANES)]
        vals = plsc.load_gather(g_v, [isrc])
        plsc.addupdate_scatter(acc_v, [idst], vals)

    @pl.when(wid < 4)
    def _():
        @plsc.parallel_loop(CH_SMALL // LANES, CH_BIG // LANES, unroll=4)
        def _(i):
            isrc = e_v[0, pl.ds(i * LANES, LANES)]
            idst = e_v[1, pl.ds(i * LANES, LANES)]
            vals = plsc.load_gather(g_v, [isrc])
            plsc.addupdate_scatter(acc_v, [idst], vals)

    pltpu.sync_copy(acc_v, out_hbm.at[wid])


def _h_body(x_ref, w_ref, h_ref):
    z = lax.dot_general(
        w_ref[...], x_ref[...], (((1,), (1,)), ((), ())),
        preferred_element_type=jnp.float32,
    )
    h_ref[...] = z[0]


def _mid_body(degp_ref, h_ref, g_ref, dis_ref):
    deg = jnp.sum(degp_ref[...], axis=0) + 1.0
    dis = lax.rsqrt(deg)
    dis_ref[...] = dis
    g = h_ref[...] * dis
    for r in range(4):
        g_ref[pl.ds(r * N_NODES, N_NODES)] = g


def _fin_body(msgp_ref, g_ref, dis_ref, b_ref, out_ref):
    p = jnp.sum(msgp_ref[...], axis=0)
    z = dis_ref[...] * (p + g_ref[pl.ds(0, N_NODES)]) + b_ref[...]
    out_ref[...] = jax.nn.sigmoid(z)


def kernel(x, edge_index, W, b):
    ei = edge_index.astype(jnp.int32)
    w2d = W.reshape((1, C_IN)).astype(jnp.float32)

    degp = _deg_kernel(ei)

    h = pl.pallas_call(
        _h_body,
        out_shape=jax.ShapeDtypeStruct((N_NODES,), jnp.float32),
    )(x, w2d)

    g4, dis = pl.pallas_call(
        _mid_body,
        out_shape=(
            jax.ShapeDtypeStruct((4 * N_NODES,), jnp.float32),
            jax.ShapeDtypeStruct((N_NODES,), jnp.float32),
        ),
    )(degp, h)

    msgp = _msg_kernel(ei, g4)

    out = pl.pallas_call(
        _fin_body,
        out_shape=jax.ShapeDtypeStruct((N_NODES,), jnp.float32),
    )(msgp, g4, dis, b)

    return out[:, None]
